# baseline (speedup 1.0000x reference)
import functools

import jax
import jax.numpy as jnp
from jax.experimental import pallas as pl
from jax.experimental.pallas import tpu as pltpu


def _cdiv(a, b):
    return -(-a // b)


def _mlp_kernel(x_ref, w1_ref, b1_ref, w2t_ref, b2_ref, o_ref):
    h = jnp.dot(x_ref[...], w1_ref[...], preferred_element_type=jnp.float32)
    h = h + b1_ref[...]
    h = jnp.maximum(h, 0.01 * h)
    y = jnp.sum(h * w2t_ref[...], axis=-1, keepdims=True) + b2_ref[...]
    o_ref[...] = y.astype(o_ref.dtype)


@functools.partial(jax.jit, static_argnames=("block_rows",))
def _run(x, w1, b1, w2, b2, *, block_rows=8192):
    B, F = x.shape
    H = w1.shape[1]
    out_dtype = x.dtype

    tb = min(block_rows, B)
    if B >= 8:
        tb = max(8, (tb // 8) * 8)
    grid = (_cdiv(B, tb),)

    out = pl.pallas_call(
        _mlp_kernel,
        out_shape=jax.ShapeDtypeStruct((B, 1), out_dtype),
        grid=grid,
        in_specs=[
            pl.BlockSpec((tb, F), lambda i: (i, 0)),
            pl.BlockSpec((F, H), lambda i: (0, 0)),
            pl.BlockSpec((1, H), lambda i: (0, 0)),
            pl.BlockSpec((1, H), lambda i: (0, 0)),
            pl.BlockSpec((1, 1), lambda i: (0, 0)),
        ],
        out_specs=pl.BlockSpec((tb, 1), lambda i: (i, 0)),
        compiler_params=pltpu.CompilerParams(
            dimension_semantics=("parallel",)),
    )(x, w1.astype(x.dtype), b1.reshape(1, H).astype(jnp.float32),
      w2.reshape(1, H).astype(jnp.float32),
      b2.reshape(1, 1).astype(jnp.float32))
    return out


def kernel(x, w1, b1, w2, b2):
    return _run(x, w1, b1, w2, b2)


# speedup vs baseline: 1.1111x; 1.1111x over previous
import functools

import jax
import jax.numpy as jnp
from jax.experimental import pallas as pl
from jax.experimental.pallas import tpu as pltpu

_PACK = 4


def _cdiv(a, b):
    return -(-a // b)


def _mlp2_kernel(x_ref, w1_ref, b1_ref, w2_ref, b2_ref, o_ref, *, tb4):
    xb = x_ref[...].reshape(tb4, _PACK * 32)
    h = jnp.dot(xb, w1_ref[...], preferred_element_type=jnp.float32)
    h = h + b1_ref[...]
    h = jnp.maximum(h, 0.01 * h)
    y = jnp.dot(h, w2_ref[...], preferred_element_type=jnp.float32)
    o_ref[...] = (y + b2_ref[...]).astype(o_ref.dtype)


def _mlp_rowwise_kernel(x_ref, w1_ref, b1_ref, w2t_ref, b2_ref, o_ref):
    h = jnp.dot(x_ref[...], w1_ref[...], preferred_element_type=jnp.float32)
    h = h + b1_ref[...]
    h = jnp.maximum(h, 0.01 * h)
    y = jnp.sum(h * w2t_ref[...], axis=-1, keepdims=True) + b2_ref[...]
    o_ref[...] = y.astype(o_ref.dtype)


def _block_diag(w, pack):
    f, h = w.shape
    out = jnp.zeros((pack * f, pack * h), w.dtype)
    for g in range(pack):
        out = out.at[g * f:(g + 1) * f, g * h:(g + 1) * h].set(w)
    return out


@functools.partial(jax.jit, static_argnames=("block_rows",))
def _run(x, w1, b1, w2, b2, *, block_rows=8192):
    B, F = x.shape
    H = w1.shape[1]
    out_dtype = x.dtype

    if B % _PACK != 0 or F != 32:
        tb = max(8, (min(B, 8192) // 8) * 8) if B >= 8 else B
        grid = (_cdiv(B, tb),)
        return pl.pallas_call(
            _mlp_rowwise_kernel,
            out_shape=jax.ShapeDtypeStruct((B, 1), out_dtype),
            grid=grid,
            in_specs=[
                pl.BlockSpec((tb, F), lambda i: (i, 0)),
                pl.BlockSpec((F, H), lambda i: (0, 0)),
                pl.BlockSpec((1, H), lambda i: (0, 0)),
                pl.BlockSpec((1, H), lambda i: (0, 0)),
                pl.BlockSpec((1, 1), lambda i: (0, 0)),
            ],
            out_specs=pl.BlockSpec((tb, 1), lambda i: (i, 0)),
            compiler_params=pltpu.CompilerParams(
                dimension_semantics=("parallel",)),
        )(x, w1.astype(x.dtype), b1.reshape(1, H).astype(jnp.float32),
          w2.reshape(1, H).astype(jnp.float32),
          b2.reshape(1, 1).astype(jnp.float32))

    B4 = B // _PACK
    x_flat = x.reshape(B4 * _PACK * F)
    w1_p = _block_diag(w1, _PACK).astype(x.dtype)
    b1_p = jnp.tile(b1.astype(jnp.float32), _PACK).reshape(1, _PACK * H)
    w2_p = _block_diag(w2.astype(jnp.float32), _PACK)
    b2_p = jnp.broadcast_to(b2.astype(jnp.float32).reshape(1, 1), (1, _PACK))

    tb4 = min(block_rows, B4)
    tb4 = max(8, (tb4 // 8) * 8)
    grid = (_cdiv(B4, tb4),)
    blk = tb4 * _PACK * F

    out = pl.pallas_call(
        functools.partial(_mlp2_kernel, tb4=tb4),
        out_shape=jax.ShapeDtypeStruct((B4, _PACK), out_dtype),
        grid=grid,
        in_specs=[
            pl.BlockSpec((blk,), lambda i: (i,)),
            pl.BlockSpec((_PACK * F, _PACK * H), lambda i: (0, 0)),
            pl.BlockSpec((1, _PACK * H), lambda i: (0, 0)),
            pl.BlockSpec((_PACK * H, _PACK), lambda i: (0, 0)),
            pl.BlockSpec((1, _PACK), lambda i: (0, 0)),
        ],
        out_specs=pl.BlockSpec((tb4, _PACK), lambda i: (i, 0)),
        compiler_params=pltpu.CompilerParams(
            dimension_semantics=("parallel",),
            vmem_limit_bytes=56 * 2**20),
    )(x_flat, w1_p, b1_p, w2_p, b2_p)
    return out.reshape(B, 1)


def kernel(x, w1, b1, w2, b2):
    return _run(x, w1, b1, w2, b2)


# speedup vs baseline: 8.6997x; 7.8300x over previous
import functools

import jax
import jax.numpy as jnp
from jax.experimental import pallas as pl
from jax.experimental.pallas import tpu as pltpu


def _cdiv(a, b):
    return -(-a // b)


def _mlp_t_kernel(xt_ref, w1t_ref, b1_ref, w2t_ref, b2_ref, o_ref):
    h = jnp.dot(w1t_ref[...], xt_ref[...],
                preferred_element_type=jnp.float32)
    h = h + b1_ref[...]
    h = jnp.maximum(h, 0.01 * h)
    y = jnp.dot(w2t_ref[...], h,
                preferred_element_type=jnp.float32)
    o_ref[...] = (y + b2_ref[...]).astype(o_ref.dtype)


@functools.partial(jax.jit, static_argnames=("block_n",))
def _run(x, w1, b1, w2, b2, *, block_n=16384):
    B, F = x.shape
    H = w1.shape[1]
    out_dtype = x.dtype

    xt = x.T
    w1t = w1.T.astype(x.dtype)
    b1c = b1.reshape(H, 1).astype(jnp.float32)
    w2t = w2.reshape(1, H).astype(jnp.float32)
    b2r = b2.reshape(1, 1).astype(jnp.float32)

    tbn = min(block_n, B)
    if B >= 128:
        tbn = max(128, (tbn // 128) * 128)
    grid = (_cdiv(B, tbn),)

    out = pl.pallas_call(
        _mlp_t_kernel,
        out_shape=jax.ShapeDtypeStruct((1, B), out_dtype),
        grid=grid,
        in_specs=[
            pl.BlockSpec((F, tbn), lambda i: (0, i)),
            pl.BlockSpec((H, F), lambda i: (0, 0)),
            pl.BlockSpec((H, 1), lambda i: (0, 0)),
            pl.BlockSpec((1, H), lambda i: (0, 0)),
            pl.BlockSpec((1, 1), lambda i: (0, 0)),
        ],
        out_specs=pl.BlockSpec((1, tbn), lambda i: (0, i)),
        compiler_params=pltpu.CompilerParams(
            dimension_semantics=("parallel",)),
    )(xt, w1t, b1c, w2t, b2r)
    return out.reshape(B, 1)


def kernel(x, w1, b1, w2, b2):
    return _run(x, w1, b1, w2, b2)


# speedup vs baseline: 10.1633x; 1.1682x over previous
import functools

import jax
import jax.numpy as jnp
from jax.experimental import pallas as pl
from jax.experimental.pallas import tpu as pltpu


def _cdiv(a, b):
    return -(-a // b)


def _mlp_t_kernel(xt_ref, w1t_ref, b1_ref, w2t_ref, b2_ref, o_ref):
    h = jnp.dot(w1t_ref[...], xt_ref[...],
                preferred_element_type=jnp.float32)
    h = h + b1_ref[...]
    h = jnp.maximum(h, 0.01 * h)
    y = jnp.dot(w2t_ref[...], h,
                preferred_element_type=jnp.float32)
    o_ref[...] = (y + b2_ref[...]).astype(o_ref.dtype)


@functools.partial(jax.jit, static_argnames=("block_n",))
def _run(x, w1, b1, w2, b2, *, block_n=32768):
    B, F = x.shape
    H = w1.shape[1]
    out_dtype = x.dtype

    xt = x.T
    w1t = w1.T.astype(x.dtype)
    b1c = b1.reshape(H, 1).astype(jnp.float32)
    w2t = w2.reshape(1, H).astype(jnp.float32)
    b2r = b2.reshape(1, 1).astype(jnp.float32)

    tbn = min(block_n, B)
    if B >= 128:
        tbn = max(128, (tbn // 128) * 128)
    grid = (_cdiv(B, tbn),)

    out = pl.pallas_call(
        _mlp_t_kernel,
        out_shape=jax.ShapeDtypeStruct((1, B), out_dtype),
        grid=grid,
        in_specs=[
            pl.BlockSpec((F, tbn), lambda i: (0, i)),
            pl.BlockSpec((H, F), lambda i: (0, 0)),
            pl.BlockSpec((H, 1), lambda i: (0, 0)),
            pl.BlockSpec((1, H), lambda i: (0, 0)),
            pl.BlockSpec((1, 1), lambda i: (0, 0)),
        ],
        out_specs=pl.BlockSpec((1, tbn), lambda i: (0, i)),
        compiler_params=pltpu.CompilerParams(
            dimension_semantics=("parallel",)),
    )(xt, w1t, b1c, w2t, b2r)
    return out.reshape(B, 1)


def kernel(x, w1, b1, w2, b2):
    return _run(x, w1, b1, w2, b2)


# speedup vs baseline: 10.4060x; 1.0239x over previous
import functools

import jax
import jax.numpy as jnp
from jax.experimental import pallas as pl
from jax.experimental.pallas import tpu as pltpu


def _cdiv(a, b):
    return -(-a // b)


def _mlp_t_kernel(xt_ref, w1t_ref, b1_ref, w2t_ref, b2_ref, o_ref):
    h = jnp.dot(w1t_ref[...], xt_ref[...],
                preferred_element_type=jnp.float32)
    h = h + b1_ref[...]
    h = jnp.maximum(h, 0.01 * h)
    y = jnp.dot(w2t_ref[...], h,
                preferred_element_type=jnp.float32)
    o_ref[...] = (y + b2_ref[...]).astype(o_ref.dtype)


@functools.partial(jax.jit, static_argnames=("block_n",))
def _run(x, w1, b1, w2, b2, *, block_n=65536):
    B, F = x.shape
    H = w1.shape[1]
    out_dtype = x.dtype

    xt = x.T
    w1t = w1.T.astype(x.dtype)
    b1c = b1.reshape(H, 1).astype(jnp.float32)
    w2t = w2.reshape(1, H).astype(jnp.float32)
    b2r = b2.reshape(1, 1).astype(jnp.float32)

    tbn = min(block_n, B)
    if B >= 128:
        tbn = max(128, (tbn // 128) * 128)
    grid = (_cdiv(B, tbn),)

    out = pl.pallas_call(
        _mlp_t_kernel,
        out_shape=jax.ShapeDtypeStruct((1, B), out_dtype),
        grid=grid,
        in_specs=[
            pl.BlockSpec((F, tbn), lambda i: (0, i)),
            pl.BlockSpec((H, F), lambda i: (0, 0)),
            pl.BlockSpec((H, 1), lambda i: (0, 0)),
            pl.BlockSpec((1, H), lambda i: (0, 0)),
            pl.BlockSpec((1, 1), lambda i: (0, 0)),
        ],
        out_specs=pl.BlockSpec((1, tbn), lambda i: (0, i)),
        compiler_params=pltpu.CompilerParams(
            dimension_semantics=("parallel",)),
    )(xt, w1t, b1c, w2t, b2r)
    return out.reshape(B, 1)


def kernel(x, w1, b1, w2, b2):
    return _run(x, w1, b1, w2, b2)


# speedup vs baseline: 11.4070x; 1.0962x over previous
import functools

import jax
import jax.numpy as jnp
from jax.experimental import pallas as pl
from jax.experimental.pallas import tpu as pltpu


def _cdiv(a, b):
    return -(-a // b)


def _mlp_t_kernel(xt_ref, w1_ref, b1r_ref, w2t_ref, b2_ref, o_ref):
    h = jax.lax.dot_general(
        w1_ref[...], xt_ref[...],
        dimension_numbers=(((0,), (0,)), ((), ())),
        preferred_element_type=jnp.float32)
    h = h + jnp.transpose(b1r_ref[...], (1, 0))
    h = jnp.maximum(h, 0.01 * h)
    y = jnp.dot(w2t_ref[...], h,
                preferred_element_type=jnp.float32)
    o_ref[...] = (y + b2_ref[...]).astype(o_ref.dtype)


@functools.partial(jax.jit, static_argnames=("block_n",))
def _run(x, w1, b1, w2, b2, *, block_n=32768):
    B, F = x.shape
    H = w1.shape[1]
    out_dtype = x.dtype

    xt = x.T
    w1c = w1.astype(jnp.float32)
    b1r = b1.reshape(1, H).astype(jnp.float32)
    w2t = w2.reshape(1, H).astype(jnp.float32)
    b2r = b2.reshape(1, 1).astype(jnp.float32)

    tbn = min(block_n, B)
    if B >= 128:
        tbn = max(128, (tbn // 128) * 128)
    grid = (_cdiv(B, tbn),)

    out = pl.pallas_call(
        _mlp_t_kernel,
        out_shape=jax.ShapeDtypeStruct((1, B), out_dtype),
        grid=grid,
        in_specs=[
            pl.BlockSpec((F, tbn), lambda i: (0, i)),
            pl.BlockSpec((F, H), lambda i: (0, 0)),
            pl.BlockSpec((1, H), lambda i: (0, 0)),
            pl.BlockSpec((1, H), lambda i: (0, 0)),
            pl.BlockSpec((1, 1), lambda i: (0, 0)),
        ],
        out_specs=pl.BlockSpec((1, tbn), lambda i: (0, i)),
        compiler_params=pltpu.CompilerParams(
            dimension_semantics=("parallel",)),
    )(xt, w1c, b1r, w2t, b2r)
    return out.reshape(B, 1)


def kernel(x, w1, b1, w2, b2):
    return _run(x, w1, b1, w2, b2)


# speedup vs baseline: 11.8309x; 1.0372x over previous
import functools

import jax
import jax.numpy as jnp
from jax.experimental import pallas as pl
from jax.experimental.pallas import tpu as pltpu


def _cdiv(a, b):
    return -(-a // b)


def _mlp_t_kernel(xt_ref, w1_ref, b1r_ref, w2t_ref, b2_ref, o_ref):
    h = jax.lax.dot_general(
        w1_ref[...], xt_ref[...],
        dimension_numbers=(((0,), (0,)), ((), ())),
        preferred_element_type=jnp.float32)
    h = h + jnp.transpose(b1r_ref[...], (1, 0))
    h = jnp.maximum(h, 0.01 * h)
    y = jnp.dot(w2t_ref[...], h,
                preferred_element_type=jnp.float32)
    o_ref[...] = (y + b2_ref[...]).astype(o_ref.dtype)


@functools.partial(jax.jit, static_argnames=("block_n",))
def _run(x, w1, b1, w2, b2, *, block_n=65536):
    B, F = x.shape
    H = w1.shape[1]
    out_dtype = x.dtype

    xt = x.T
    w1c = w1.astype(jnp.float32)
    b1r = b1.reshape(1, H).astype(jnp.float32)
    w2t = w2.reshape(1, H).astype(jnp.float32)
    b2r = b2.reshape(1, 1).astype(jnp.float32)

    tbn = min(block_n, B)
    if B >= 128:
        tbn = max(128, (tbn // 128) * 128)
    grid = (_cdiv(B, tbn),)

    out = pl.pallas_call(
        _mlp_t_kernel,
        out_shape=jax.ShapeDtypeStruct((1, B), out_dtype),
        grid=grid,
        in_specs=[
            pl.BlockSpec((F, tbn), lambda i: (0, i)),
            pl.BlockSpec((F, H), lambda i: (0, 0)),
            pl.BlockSpec((1, H), lambda i: (0, 0)),
            pl.BlockSpec((1, H), lambda i: (0, 0)),
            pl.BlockSpec((1, 1), lambda i: (0, 0)),
        ],
        out_specs=pl.BlockSpec((1, tbn), lambda i: (0, i)),
        compiler_params=pltpu.CompilerParams(
            dimension_semantics=("parallel",)),
    )(xt, w1c, b1r, w2t, b2r)
    return out.reshape(B, 1)


def kernel(x, w1, b1, w2, b2):
    return _run(x, w1, b1, w2, b2)
